# Initial kernel scaffold; baseline (speedup 1.0000x reference)
import jax
import jax.numpy as jnp
from jax.experimental import pallas as pl


def kernel(data, mask, rope_cos, rope_sin, attn_norm_w, ffn_norm_w, wq, wk, wv, wo, gate_w, w1, w2, w3, lora_a1, lora_b1, lora_a3, lora_b3, lora_a2, lora_b2):
    raise NotImplementedError("write your pallas kernel here")


# speedup vs baseline: 2.2760x; 2.2760x over previous
import jax
import jax.numpy as jnp
from jax.experimental import pallas as pl
from jax.experimental.pallas import tpu as pltpu

B, S, D = 1, 2048, 1024
H, KVH, DH = 16, 8, 64
E, K, FF, R = 8, 2, 2816, 16
SCALE = 32.0 / 16.0
EPS = 1e-5
SBLK = 256
NBLK = S // SBLK
ER = E * R
NREP = H // KVH


def _rmsnorm(x, w):
    return x * jax.lax.rsqrt(jnp.mean(x * x, axis=-1, keepdims=True) + EPS) * w


def _rot_half_tiled(x, nh):
    pieces = []
    for h in range(nh):
        x1 = x[:, h * DH : h * DH + DH // 2]
        x2 = x[:, h * DH + DH // 2 : (h + 1) * DH]
        pieces.append(-x2)
        pieces.append(x1)
    return jnp.concatenate(pieces, axis=1)


def _qkv_kernel(x_ref, cosq_ref, sinq_ref, cosk_ref, sink_ref, wn_ref,
                wq_ref, wk_ref, wv_ref, q_ref, k_ref, v_ref):
    h = _rmsnorm(x_ref[...], wn_ref[...])
    q = jnp.dot(h, wq_ref[...], preferred_element_type=jnp.float32)
    k = jnp.dot(h, wk_ref[...], preferred_element_type=jnp.float32)
    v = jnp.dot(h, wv_ref[...], preferred_element_type=jnp.float32)
    q_ref[...] = q * cosq_ref[...] + _rot_half_tiled(q, H) * sinq_ref[...]
    k_ref[...] = k * cosk_ref[...] + _rot_half_tiled(k, KVH) * sink_ref[...]
    v_ref[...] = v


def _attn_kernel(q_ref, k_ref, v_ref, mask_ref, o_ref):
    q = q_ref[0]
    k = k_ref[0]
    s = jax.lax.dot_general(q, k, (((1,), (1,)), ((), ())),
                            preferred_element_type=jnp.float32)
    s = s * (1.0 / (DH ** 0.5)) + mask_ref[...]
    m = jnp.max(s, axis=1, keepdims=True)
    e = jnp.exp(s - m)
    p = e / jnp.sum(e, axis=1, keepdims=True)
    o_ref[0] = jnp.dot(p, v_ref[0], preferred_element_type=jnp.float32)


def _post_kernel(attn_ref, x_ref, wo_ref, wn_ref, gw_ref, d2_ref, sn_ref, lg_ref):
    o = jnp.dot(attn_ref[...], wo_ref[...], preferred_element_type=jnp.float32)
    d2 = x_ref[...] + o
    sn = _rmsnorm(d2, wn_ref[...])
    d2_ref[...] = d2
    sn_ref[...] = sn
    lg_ref[...] = jnp.dot(sn, gw_ref[...], preferred_element_type=jnp.float32)


def _route_kernel(lg_ref, oh0_ref, oh1_ref, wt0_ref, wt1_ref):
    lg = lg_ref[...]
    cm = jnp.max(lg, axis=0, keepdims=True)
    ex = jnp.exp(lg - cm)
    p = ex / jnp.sum(ex, axis=0, keepdims=True)
    idx = jax.lax.broadcasted_iota(jnp.int32, (S, E), 1)
    v0 = jnp.max(p, axis=1, keepdims=True)
    i0 = jnp.min(jnp.where(p >= v0, idx, E), axis=1, keepdims=True)
    oh0 = idx == i0
    pm = jnp.where(oh0, -jnp.inf, p)
    v1 = jnp.max(pm, axis=1, keepdims=True)
    i1 = jnp.min(jnp.where(pm >= v1, idx, E), axis=1, keepdims=True)
    oh1 = idx == i1
    tot = v0 + v1
    oh0_ref[...] = oh0.astype(jnp.float32)
    oh1_ref[...] = oh1.astype(jnp.float32)
    wt0_ref[...] = jnp.broadcast_to(v0 / tot, (S, E))
    wt1_ref[...] = jnp.broadcast_to(v1 / tot, (S, E))


def _moe_kernel(d2_ref, sn_ref, oh0_ref, oh1_ref, wt0_ref, wt1_ref,
                w1_ref, w3_ref, w2_ref, a1t_ref, a3t_ref, b1_ref, b3_ref,
                a2t_ref, b2_ref, out_ref):
    sn = sn_ref[...]
    c1 = jnp.dot(sn, w1_ref[...], preferred_element_type=jnp.float32)
    c3 = jnp.dot(sn, w3_ref[...], preferred_element_type=jnp.float32)
    z1 = jnp.dot(c1, a1t_ref[...], preferred_element_type=jnp.float32)
    z3 = jnp.dot(c3, a3t_ref[...], preferred_element_type=jnp.float32)
    re = jax.lax.broadcasted_iota(jnp.int32, (E, ER), 0)
    le = jax.lax.broadcasted_iota(jnp.int32, (E, ER), 1) // R
    expm = (re == le).astype(jnp.float32)
    acc = d2_ref[...]
    for oh_ref, wt_ref in ((oh0_ref, wt0_ref), (oh1_ref, wt1_ref)):
        m = jnp.dot(oh_ref[...], expm, preferred_element_type=jnp.float32)
        corr1 = jnp.dot(z1 * m, b1_ref[...], preferred_element_type=jnp.float32) * SCALE
        corr3 = jnp.dot(z3 * m, b3_ref[...], preferred_element_type=jnp.float32) * SCALE
        g = c1 + corr1
        u = (g * jax.lax.logistic(g)) * (c3 + corr3)
        hs = jnp.dot(u, w2_ref[...], preferred_element_type=jnp.float32)
        y2 = jnp.dot(hs, a2t_ref[...], preferred_element_type=jnp.float32)
        hs = hs + jnp.dot(y2 * m, b2_ref[...], preferred_element_type=jnp.float32) * SCALE
        acc = acc + hs * wt_ref[...][:, :1]
    out_ref[...] = acc


def kernel(data, mask, rope_cos, rope_sin, attn_norm_w, ffn_norm_w, wq, wk, wv,
           wo, gate_w, w1, w2, w3, lora_a1, lora_b1, lora_a3, lora_b3,
           lora_a2, lora_b2):
    x = data[0]
    cosq = jnp.tile(rope_cos, (1, H))
    sinq = jnp.tile(rope_sin, (1, H))
    cosk = jnp.tile(rope_cos, (1, KVH))
    sink = jnp.tile(rope_sin, (1, KVH))
    anw = attn_norm_w.reshape(1, D)
    fnw = ffn_norm_w.reshape(1, D)
    a1t = lora_a1.reshape(ER, FF).T
    a3t = lora_a3.reshape(ER, FF).T
    b1 = lora_b1.transpose(0, 2, 1).reshape(ER, FF)
    b3 = lora_b3.transpose(0, 2, 1).reshape(ER, FF)
    a2t = lora_a2.reshape(ER, D).T
    b2 = lora_b2.transpose(0, 2, 1).reshape(ER, D)
    f32 = jnp.float32

    full = lambda shape: pl.BlockSpec(shape, lambda i: (0,) * len(shape))
    rows = lambda cols: pl.BlockSpec((SBLK, cols), lambda i: (i, 0))

    q, k, v = pl.pallas_call(
        _qkv_kernel,
        grid=(NBLK,),
        in_specs=[rows(D), rows(H * DH), rows(H * DH), rows(KVH * DH),
                  rows(KVH * DH), full((1, D)), full((D, H * DH)),
                  full((D, KVH * DH)), full((D, KVH * DH))],
        out_specs=[rows(H * DH), rows(KVH * DH), rows(KVH * DH)],
        out_shape=[jax.ShapeDtypeStruct((S, H * DH), f32),
                   jax.ShapeDtypeStruct((S, KVH * DH), f32),
                   jax.ShapeDtypeStruct((S, KVH * DH), f32)],
    )(x, cosq, sinq, cosk, sink, anw, wq, wk, wv)

    q3 = q.reshape(S, H, DH).transpose(1, 0, 2)
    k3 = k.reshape(S, KVH, DH).transpose(1, 0, 2)
    v3 = v.reshape(S, KVH, DH).transpose(1, 0, 2)
    attn3 = pl.pallas_call(
        _attn_kernel,
        grid=(H, NBLK),
        in_specs=[pl.BlockSpec((1, SBLK, DH), lambda h, i: (h, i, 0)),
                  pl.BlockSpec((1, S, DH), lambda h, i: (h // NREP, 0, 0)),
                  pl.BlockSpec((1, S, DH), lambda h, i: (h // NREP, 0, 0)),
                  pl.BlockSpec((SBLK, S), lambda h, i: (i, 0))],
        out_specs=pl.BlockSpec((1, SBLK, DH), lambda h, i: (h, i, 0)),
        out_shape=jax.ShapeDtypeStruct((H, S, DH), f32),
    )(q3, k3, v3, mask)
    attn = attn3.transpose(1, 0, 2).reshape(S, H * DH)

    d2, sn, lg = pl.pallas_call(
        _post_kernel,
        grid=(NBLK,),
        in_specs=[rows(H * DH), rows(D), full((H * DH, D)), full((1, D)),
                  full((D, E))],
        out_specs=[rows(D), rows(D), rows(E)],
        out_shape=[jax.ShapeDtypeStruct((S, D), f32),
                   jax.ShapeDtypeStruct((S, D), f32),
                   jax.ShapeDtypeStruct((S, E), f32)],
    )(attn, x, wo, fnw, gate_w)

    oh0, oh1, wt0, wt1 = pl.pallas_call(
        _route_kernel,
        out_shape=[jax.ShapeDtypeStruct((S, E), f32)] * 4,
    )(lg)

    out = pl.pallas_call(
        _moe_kernel,
        grid=(NBLK,),
        in_specs=[rows(D), rows(D), rows(E), rows(E), rows(E), rows(E),
                  full((D, FF)), full((D, FF)), full((FF, D)),
                  full((FF, ER)), full((FF, ER)), full((ER, FF)),
                  full((ER, FF)), full((D, ER)), full((ER, D))],
        out_specs=rows(D),
        out_shape=jax.ShapeDtypeStruct((S, D), f32),
        compiler_params=pltpu.CompilerParams(vmem_limit_bytes=100 * 1024 * 1024),
    )(d2, sn, oh0, oh1, wt0, wt1, w1, w3, w2, a1t, a3t, b1, b3, a2t, b2)

    return out[None]


# speedup vs baseline: 2.8523x; 1.2532x over previous
import jax
import jax.numpy as jnp
from jax.experimental import pallas as pl
from jax.experimental.pallas import tpu as pltpu

B, S, D = 1, 2048, 1024
H, KVH, DH = 16, 8, 64
E, K, FF, R = 8, 2, 2816, 16
SCALE = 32.0 / 16.0
EPS = 1e-5
SBLK = 256
NBLK = S // SBLK
ABLK = 512
NABLK = S // ABLK
ER = E * R
NREP = H // KVH


def _rmsnorm(x, w):
    return x * jax.lax.rsqrt(jnp.mean(x * x, axis=-1, keepdims=True) + EPS) * w


def _qkv_kernel(x_ref, cos_ref, sin_ref, wn_ref, wq_ref, wk_ref, wv_ref,
                q_ref, k_ref, v_ref):
    h = _rmsnorm(x_ref[...], wn_ref[...])
    q = jnp.dot(h, wq_ref[...], preferred_element_type=jnp.float32)
    k = jnp.dot(h, wk_ref[...], preferred_element_type=jnp.float32)
    v = jnp.dot(h, wv_ref[...], preferred_element_type=jnp.float32)
    cos = cos_ref[...]
    sin = sin_ref[...]
    hw = DH // 2
    for i in range(H):
        qh = q[:, i * DH:(i + 1) * DH]
        rot = jnp.concatenate([-qh[:, hw:], qh[:, :hw]], axis=1)
        q_ref[i] = (qh * cos + rot * sin) * (1.0 / (DH ** 0.5))
    for i in range(KVH):
        kh = k[:, i * DH:(i + 1) * DH]
        rot = jnp.concatenate([-kh[:, hw:], kh[:, :hw]], axis=1)
        k_ref[i] = kh * cos + rot * sin
        v_ref[i] = v[:, i * DH:(i + 1) * DH]


def _attn_kernel(q_ref, k_ref, v_ref, mask_ref, o_ref):
    i = pl.program_id(1)
    q = q_ref[0]
    k = k_ref[0]
    s = jax.lax.dot_general(q, k, (((1,), (1,)), ((), ())),
                            preferred_element_type=jnp.float32)
    s = s + mask_ref[pl.ds(i * ABLK, ABLK), :]
    m = jnp.max(s, axis=1, keepdims=True)
    e = jnp.exp(s - m)
    o = jnp.dot(e, v_ref[0], preferred_element_type=jnp.float32)
    o_ref[0] = o / jnp.sum(e, axis=1, keepdims=True)


def _post_kernel(attn_ref, x_ref, wo_ref, wn_ref, gw_ref, d2_ref, sn_ref, lg_ref):
    o = jnp.dot(attn_ref[0], wo_ref[pl.ds(0, DH), :],
                preferred_element_type=jnp.float32)
    for i in range(1, H):
        o = o + jnp.dot(attn_ref[i], wo_ref[pl.ds(i * DH, DH), :],
                        preferred_element_type=jnp.float32)
    d2 = x_ref[...] + o
    sn = _rmsnorm(d2, wn_ref[...])
    d2_ref[...] = d2
    sn_ref[...] = sn
    lg_ref[...] = jnp.dot(sn, gw_ref[...], preferred_element_type=jnp.float32)


def _route_kernel(lg_ref, oh0_ref, oh1_ref, wt0_ref, wt1_ref):
    lg = lg_ref[...]
    cm = jnp.max(lg, axis=0, keepdims=True)
    ex = jnp.exp(lg - cm)
    p = ex / jnp.sum(ex, axis=0, keepdims=True)
    idx = jax.lax.broadcasted_iota(jnp.int32, (S, E), 1)
    v0 = jnp.max(p, axis=1, keepdims=True)
    i0 = jnp.min(jnp.where(p >= v0, idx, E), axis=1, keepdims=True)
    oh0 = idx == i0
    pm = jnp.where(oh0, -jnp.inf, p)
    v1 = jnp.max(pm, axis=1, keepdims=True)
    i1 = jnp.min(jnp.where(pm >= v1, idx, E), axis=1, keepdims=True)
    oh1 = idx == i1
    tot = v0 + v1
    oh0_ref[...] = oh0.astype(jnp.float32)
    oh1_ref[...] = oh1.astype(jnp.float32)
    wt0_ref[...] = jnp.broadcast_to(v0 / tot, (S, E))
    wt1_ref[...] = jnp.broadcast_to(v1 / tot, (S, E))


def _moe_kernel(d2_ref, sn_ref, oh0_ref, oh1_ref, wt0_ref, wt1_ref,
                w1_ref, w3_ref, w2_ref, a1t_ref, a3t_ref, b1_ref, b3_ref,
                a2t_ref, b2_ref, out_ref):
    sn = sn_ref[...]
    c1 = jnp.dot(sn, w1_ref[...], preferred_element_type=jnp.float32)
    c3 = jnp.dot(sn, w3_ref[...], preferred_element_type=jnp.float32)
    z1 = jnp.dot(c1, a1t_ref[...], preferred_element_type=jnp.float32)
    z3 = jnp.dot(c3, a3t_ref[...], preferred_element_type=jnp.float32)
    re = jax.lax.broadcasted_iota(jnp.int32, (E, ER), 0)
    le = jax.lax.broadcasted_iota(jnp.int32, (E, ER), 1) // R
    expm = (re == le).astype(jnp.float32)
    acc = d2_ref[...]
    for oh_ref, wt_ref in ((oh0_ref, wt0_ref), (oh1_ref, wt1_ref)):
        m = jnp.dot(oh_ref[...], expm, preferred_element_type=jnp.float32)
        corr1 = jnp.dot(z1 * m, b1_ref[...], preferred_element_type=jnp.float32) * SCALE
        corr3 = jnp.dot(z3 * m, b3_ref[...], preferred_element_type=jnp.float32) * SCALE
        g = c1 + corr1
        u = (g * jax.lax.logistic(g)) * (c3 + corr3)
        hs = jnp.dot(u, w2_ref[...], preferred_element_type=jnp.float32)
        y2 = jnp.dot(hs, a2t_ref[...], preferred_element_type=jnp.float32)
        hs = hs + jnp.dot(y2 * m, b2_ref[...], preferred_element_type=jnp.float32) * SCALE
        acc = acc + hs * wt_ref[...][:, :1]
    out_ref[...] = acc


def kernel(data, mask, rope_cos, rope_sin, attn_norm_w, ffn_norm_w, wq, wk, wv,
           wo, gate_w, w1, w2, w3, lora_a1, lora_b1, lora_a3, lora_b3,
           lora_a2, lora_b2):
    x = data[0]
    anw = attn_norm_w.reshape(1, D)
    fnw = ffn_norm_w.reshape(1, D)
    a1t = lora_a1.reshape(ER, FF).T
    a3t = lora_a3.reshape(ER, FF).T
    b1 = lora_b1.transpose(0, 2, 1).reshape(ER, FF)
    b3 = lora_b3.transpose(0, 2, 1).reshape(ER, FF)
    a2t = lora_a2.reshape(ER, D).T
    b2 = lora_b2.transpose(0, 2, 1).reshape(ER, D)
    f32 = jnp.float32

    full = lambda shape: pl.BlockSpec(shape, lambda i: (0,) * len(shape))
    rows = lambda cols: pl.BlockSpec((SBLK, cols), lambda i: (i, 0))

    q3, k3, v3 = pl.pallas_call(
        _qkv_kernel,
        grid=(NBLK,),
        in_specs=[rows(D), rows(DH), rows(DH), full((1, D)),
                  full((D, H * DH)), full((D, KVH * DH)), full((D, KVH * DH))],
        out_specs=[pl.BlockSpec((H, SBLK, DH), lambda i: (0, i, 0)),
                   pl.BlockSpec((KVH, SBLK, DH), lambda i: (0, i, 0)),
                   pl.BlockSpec((KVH, SBLK, DH), lambda i: (0, i, 0))],
        out_shape=[jax.ShapeDtypeStruct((H, S, DH), f32),
                   jax.ShapeDtypeStruct((KVH, S, DH), f32),
                   jax.ShapeDtypeStruct((KVH, S, DH), f32)],
    )(x, rope_cos, rope_sin, anw, wq, wk, wv)

    attn3 = pl.pallas_call(
        _attn_kernel,
        grid=(H, NABLK),
        in_specs=[pl.BlockSpec((1, ABLK, DH), lambda h, i: (h, i, 0)),
                  pl.BlockSpec((1, S, DH), lambda h, i: (h // NREP, 0, 0)),
                  pl.BlockSpec((1, S, DH), lambda h, i: (h // NREP, 0, 0)),
                  pl.BlockSpec((S, S), lambda h, i: (0, 0))],
        out_specs=pl.BlockSpec((1, ABLK, DH), lambda h, i: (h, i, 0)),
        out_shape=jax.ShapeDtypeStruct((H, S, DH), f32),
        compiler_params=pltpu.CompilerParams(vmem_limit_bytes=100 * 1024 * 1024),
    )(q3, k3, v3, mask)

    d2, sn, lg = pl.pallas_call(
        _post_kernel,
        grid=(NBLK,),
        in_specs=[pl.BlockSpec((H, SBLK, DH), lambda i: (0, i, 0)),
                  rows(D), full((H * DH, D)), full((1, D)), full((D, E))],
        out_specs=[rows(D), rows(D), rows(E)],
        out_shape=[jax.ShapeDtypeStruct((S, D), f32),
                   jax.ShapeDtypeStruct((S, D), f32),
                   jax.ShapeDtypeStruct((S, E), f32)],
    )(attn3, x, wo, fnw, gate_w)

    oh0, oh1, wt0, wt1 = pl.pallas_call(
        _route_kernel,
        out_shape=[jax.ShapeDtypeStruct((S, E), f32)] * 4,
    )(lg)

    out = pl.pallas_call(
        _moe_kernel,
        grid=(NBLK,),
        in_specs=[rows(D), rows(D), rows(E), rows(E), rows(E), rows(E),
                  full((D, FF)), full((D, FF)), full((FF, D)),
                  full((FF, ER)), full((FF, ER)), full((ER, FF)),
                  full((ER, FF)), full((D, ER)), full((ER, D))],
        out_specs=rows(D),
        out_shape=jax.ShapeDtypeStruct((S, D), f32),
        compiler_params=pltpu.CompilerParams(vmem_limit_bytes=100 * 1024 * 1024),
    )(d2, sn, oh0, oh1, wt0, wt1, w1, w3, w2, a1t, a3t, b1, b3, a2t, b2)

    return out[None]
